# baseline (speedup 1.0000x reference)
import functools

import jax
import jax.numpy as jnp
from jax import lax
from jax.experimental import pallas as pl
from jax.experimental.pallas import tpu as pltpu

BATCH = 1024
SEQ = 50
VOCAB = 1000
BB = 8
GRID = BATCH // BB


def _onehot_tc_body(ann_ref, out_ref):
    ann = ann_ref[...]
    cols = lax.broadcasted_iota(jnp.int32, (BB, SEQ, VOCAB), 2)
    hit = (cols == ann[:, :, None]) & (ann[:, :, None] != 0)
    out_ref[...] = hit.astype(jnp.float32)


@jax.jit
def _onehot_tc(ann):
    return pl.pallas_call(
        _onehot_tc_body,
        grid=(GRID,),
        in_specs=[pl.BlockSpec((BB, SEQ), lambda i: (i, 0))],
        out_specs=pl.BlockSpec((BB, SEQ, VOCAB), lambda i: (i, 0, 0)),
        out_shape=jax.ShapeDtypeStruct((BATCH, SEQ, VOCAB), jnp.float32),
        compiler_params=pltpu.CompilerParams(
            dimension_semantics=("arbitrary",),
        ),
    )(ann)


def kernel(annotation, alignment, W):
    del alignment, W
    return _onehot_tc(annotation.astype(jnp.int32))


# speedup vs baseline: 1.1341x; 1.1341x over previous
import functools

import jax
import jax.numpy as jnp
from jax import lax
from jax.experimental import pallas as pl
from jax.experimental.pallas import tpu as pltpu

BATCH = 1024
SEQ = 50
VOCAB = 1000
BB = 32
GRID = BATCH // BB


def _onehot_tc_body(ann_ref, out_ref):
    ann = ann_ref[...]
    cols = lax.broadcasted_iota(jnp.int32, (BB, SEQ, VOCAB), 2)
    hit = (cols == ann[:, :, None]) & (ann[:, :, None] != 0)
    out_ref[...] = hit.astype(jnp.float32)


@jax.jit
def _onehot_tc(ann):
    return pl.pallas_call(
        _onehot_tc_body,
        grid=(GRID,),
        in_specs=[pl.BlockSpec((BB, SEQ), lambda i: (i, 0))],
        out_specs=pl.BlockSpec((BB, SEQ, VOCAB), lambda i: (i, 0, 0)),
        out_shape=jax.ShapeDtypeStruct((BATCH, SEQ, VOCAB), jnp.float32),
        compiler_params=pltpu.CompilerParams(
            dimension_semantics=("arbitrary",),
        ),
    )(ann)


def kernel(annotation, alignment, W):
    del alignment, W
    return _onehot_tc(annotation.astype(jnp.int32))
